# baseline (speedup 1.0000x reference)
import functools

import jax
import jax.numpy as jnp
from jax import lax
from jax.experimental import pallas as pl
from jax.experimental.pallas import tpu as pltpu
from jax.experimental.pallas import tpu_sc as plsc

N_ROWS = 32768
N_COLS = 128
TOPK = 256
CHUNK = 2048
N_CHUNKS = N_ROWS // CHUNK


def _roll(x, shift):
    return pltpu.roll(x, shift % x.shape[0], 0)


def _cmpx(v, i, dlog2, desc):
    d = 1 << dlog2
    vd = _roll(v, -d)
    vu = _roll(v, d)
    ii = _roll(i, -d)
    iu = _roll(i, d)
    rows = lax.broadcasted_iota(jnp.int32, (v.shape[0], 1), 0)
    low = ((rows >> dlog2) & 1) == 0
    pv = jnp.where(low, vd, vu)
    pi = jnp.where(low, ii, iu)
    self_gt = (v > pv) | ((v == pv) & (i < pi))
    keep_self = (low == desc) == self_gt
    return jnp.where(keep_self, v, pv), jnp.where(keep_self, i, pi)


def _cmpx_split(v, i, dlog2, s, row_offset=0, fd=None):
    d = 1 << dlog2
    r = v.shape[0]
    nb = r // (2 * d)
    v4 = v.reshape(nb, 2, d, N_COLS)
    i4 = i.reshape(nb, 2, d, N_COLS)
    av, bv = v4[:, 0], v4[:, 1]
    ai, bi = i4[:, 0], i4[:, 1]
    gt = (av > bv) | ((av == bv) & (ai < bi))
    mxv = jnp.where(gt, av, bv)
    mnv = jnp.where(gt, bv, av)
    mxi = jnp.where(gt, ai, bi)
    mni = jnp.where(gt, bi, ai)
    if fd is None:
        blk = lax.broadcasted_iota(jnp.int32, (nb, 1, 1), 0)
        desc = (((blk * 2 * d + row_offset) >> s) & 1) == 0
    else:
        desc = jnp.full((nb, 1, 1), fd)
    o0v = jnp.where(desc, mxv, mnv)
    o1v = jnp.where(desc, mnv, mxv)
    o0i = jnp.where(desc, mxi, mni)
    o1i = jnp.where(desc, mni, mxi)
    v2 = jnp.stack([o0v, o1v], axis=1).reshape(r, N_COLS)
    i2 = jnp.stack([o0i, o1i], axis=1).reshape(r, N_COLS)
    return v2, i2


def _halve(v, i, final_desc=None):
    av, bv = v[:TOPK], v[TOPK:]
    ai, bi = i[:TOPK], i[TOPK:]
    self_gt = (av > bv) | ((av == bv) & (ai < bi))
    v2 = jnp.where(self_gt, av, bv)
    i2 = jnp.where(self_gt, ai, bi)
    desc = jnp.full((TOPK, 1), final_desc)
    for j in range(7, 2, -1):
        v2, i2 = _cmpx_split(v2, i2, j, 8, fd=final_desc)
    for j in range(2, -1, -1):
        v2, i2 = _cmpx(v2, i2, j, desc)
    return v2, i2


def _sort_block(v, c):
    i = lax.broadcasted_iota(jnp.int32, (TOPK, N_COLS), 0) + c * TOPK
    rows = lax.broadcasted_iota(jnp.int32, (TOPK, 1), 0) + c * TOPK
    for s in range(1, 9):
        desc = ((rows >> s) & 1) == 0
        for j in range(s - 1, -1, -1):
            if j >= 3:
                v, i = _cmpx_split(v, i, j, s, row_offset=c * TOPK)
            else:
                v, i = _cmpx(v, i, j, desc)
    return v, i


def _merge_tail(v, i, i_out):
    while v.shape[0] > TOPK:
        r = v.shape[0]
        nb = r // (2 * TOPK)
        parts = []
        for k in range(nb):
            blk_v = v[k * 2 * TOPK:(k + 1) * 2 * TOPK]
            blk_i = i[k * 2 * TOPK:(k + 1) * 2 * TOPK]
            parts.append(_halve(blk_v, blk_i, final_desc=(k % 2) == 0))
        if nb > 1:
            v = jnp.concatenate([p[0] for p in parts], axis=0)
            i = jnp.concatenate([p[1] for p in parts], axis=0)
        else:
            v, i = parts[0]
    i_out[...] = i


def _fused_body(h_ref, i_out, sv_ref, si_ref):
    c = pl.program_id(0)
    nblk = N_ROWS // TOPK

    @pl.when(c < nblk)
    def _sort_step():
        v, i = _sort_block(h_ref[...], c)
        sv_ref[pl.ds(c * TOPK, TOPK), :] = v
        si_ref[pl.ds(c * TOPK, TOPK), :] = i

    @pl.when(c == nblk)
    def _merge_step():
        _merge_tail(sv_ref[...], si_ref[...], i_out)


def _topk_indices(h):
    nblk = N_ROWS // TOPK
    return pl.pallas_call(
        _fused_body,
        grid=(nblk + 1,),
        in_specs=[
            pl.BlockSpec((TOPK, N_COLS),
                         lambda c: (jnp.minimum(c, N_ROWS // TOPK - 1), 0)),
        ],
        out_specs=pl.BlockSpec((TOPK, N_COLS), lambda c: (0, 0)),
        out_shape=jax.ShapeDtypeStruct((TOPK, N_COLS), jnp.int32),
        scratch_shapes=[
            pltpu.VMEM((N_ROWS, N_COLS), jnp.float32),
            pltpu.VMEM((N_ROWS, N_COLS), jnp.int32),
        ],
    )(h)


def _sc_gather(h, idx):
    info = plsc.get_sparse_core_info()
    nc, ns = info.num_cores, info.num_subcores
    nw = nc * ns
    rows_per_w = TOPK // nw

    @functools.partial(
        pl.kernel,
        mesh=plsc.VectorSubcoreMesh(core_axis_name="c", subcore_axis_name="s"),
        out_type=jax.ShapeDtypeStruct((TOPK * N_COLS, N_COLS), jnp.float32),
        scratch_types=[
            pltpu.VMEM((rows_per_w, N_COLS), jnp.int32),
            pltpu.VMEM((N_COLS, N_COLS), jnp.float32),
            pltpu.SemaphoreType.DMA,
        ],
    )
    def gk(h_hbm, idx_hbm, out_hbm, idx_v, rows_v, sem):
        wid = lax.axis_index("s") * nc + lax.axis_index("c")
        pltpu.sync_copy(idx_hbm.at[pl.ds(wid * rows_per_w, rows_per_w)], idx_v)
        for j in range(rows_per_w):
            pltpu.async_copy(h_hbm.at[idx_v.at[j]], rows_v, sem).wait()
            pltpu.sync_copy(
                rows_v,
                out_hbm.at[pl.ds((wid * rows_per_w + j) * N_COLS, N_COLS)],
            )

    return gk(h, idx)


def kernel(h):
    idx = _topk_indices(h)
    flat = _sc_gather(h, idx)
    return flat.reshape(TOPK, N_COLS, N_COLS)


# speedup vs baseline: 1.1287x; 1.1287x over previous
import functools

import jax
import jax.numpy as jnp
from jax import lax
from jax.experimental import pallas as pl
from jax.experimental.pallas import tpu as pltpu
from jax.experimental.pallas import tpu_sc as plsc

N_ROWS = 32768
N_COLS = 128
TOPK = 256
CHUNK = 2048
N_CHUNKS = N_ROWS // CHUNK


def _roll(x, shift):
    return pltpu.roll(x, shift % x.shape[0], 0)


def _cmpx(v, i, dlog2, desc):
    d = 1 << dlog2
    vd = _roll(v, -d)
    vu = _roll(v, d)
    ii = _roll(i, -d)
    iu = _roll(i, d)
    rows = lax.broadcasted_iota(jnp.int32, (v.shape[0], 1), 0)
    low = ((rows >> dlog2) & 1) == 0
    pv = jnp.where(low, vd, vu)
    pi = jnp.where(low, ii, iu)
    self_gt = (v > pv) | ((v == pv) & (i < pi))
    keep_self = (low == desc) == self_gt
    return jnp.where(keep_self, v, pv), jnp.where(keep_self, i, pi)


def _cmpx_split(v, i, dlog2, s, row_offset=0, fd=None):
    d = 1 << dlog2
    r = v.shape[0]
    nb = r // (2 * d)
    v4 = v.reshape(nb, 2, d, N_COLS)
    i4 = i.reshape(nb, 2, d, N_COLS)
    av, bv = v4[:, 0], v4[:, 1]
    ai, bi = i4[:, 0], i4[:, 1]
    gt = (av > bv) | ((av == bv) & (ai < bi))
    mxv = jnp.where(gt, av, bv)
    mnv = jnp.where(gt, bv, av)
    mxi = jnp.where(gt, ai, bi)
    mni = jnp.where(gt, bi, ai)
    if fd is None:
        blk = lax.broadcasted_iota(jnp.int32, (nb, 1, 1), 0)
        desc = (((blk * 2 * d + row_offset) >> s) & 1) == 0
    else:
        desc = jnp.full((nb, 1, 1), fd)
    o0v = jnp.where(desc, mxv, mnv)
    o1v = jnp.where(desc, mnv, mxv)
    o0i = jnp.where(desc, mxi, mni)
    o1i = jnp.where(desc, mni, mxi)
    v2 = jnp.stack([o0v, o1v], axis=1).reshape(r, N_COLS)
    i2 = jnp.stack([o0i, o1i], axis=1).reshape(r, N_COLS)
    return v2, i2


def _halve(v, i, final_desc=None):
    av, bv = v[:TOPK], v[TOPK:]
    ai, bi = i[:TOPK], i[TOPK:]
    self_gt = (av > bv) | ((av == bv) & (ai < bi))
    v2 = jnp.where(self_gt, av, bv)
    i2 = jnp.where(self_gt, ai, bi)
    desc = jnp.full((TOPK, 1), final_desc)
    for j in range(7, 2, -1):
        v2, i2 = _cmpx_split(v2, i2, j, 8, fd=final_desc)
    for j in range(2, -1, -1):
        v2, i2 = _cmpx(v2, i2, j, desc)
    return v2, i2


def _sort_body(h_ref, v_out, i_out):
    c = pl.program_id(0)
    v = h_ref[...]
    i = lax.broadcasted_iota(jnp.int32, (TOPK, N_COLS), 0) + c * TOPK
    rows = lax.broadcasted_iota(jnp.int32, (TOPK, 1), 0) + c * TOPK
    for s in range(1, 9):
        desc = ((rows >> s) & 1) == 0
        for j in range(s - 1, -1, -1):
            if j >= 3:
                v, i = _cmpx_split(v, i, j, s, row_offset=c * TOPK)
            else:
                v, i = _cmpx(v, i, j, desc)
    v_out[...] = v
    i_out[...] = i


def _merge_tail_body(v_ref, i_ref, i_out):
    v = v_ref[...]
    i = i_ref[...]
    while v.shape[0] > TOPK:
        r = v.shape[0]
        nb = r // (2 * TOPK)
        av, bv = [], []
        parts = []
        for k in range(nb):
            blk_v = v[k * 2 * TOPK:(k + 1) * 2 * TOPK]
            blk_i = i[k * 2 * TOPK:(k + 1) * 2 * TOPK]
            parts.append(_halve(blk_v, blk_i, final_desc=(k % 2) == 0))
        if nb > 1:
            v = jnp.concatenate([p[0] for p in parts], axis=0)
            i = jnp.concatenate([p[1] for p in parts], axis=0)
        else:
            v, i = parts[0]
    i_out[...] = i


def _pairmerge_body(v_ref, i_ref, v_out, i_out):
    c = pl.program_id(0)
    v, i = _halve(v_ref[...], i_ref[...], final_desc=(c % 2) == 0)
    v_out[...] = v
    i_out[...] = i


def _topk_indices(h):
    sv, si = pl.pallas_call(
        _sort_body,
        grid=(N_ROWS // TOPK,),
        in_specs=[pl.BlockSpec((TOPK, N_COLS), lambda c: (c, 0))],
        out_specs=[
            pl.BlockSpec((TOPK, N_COLS), lambda c: (c, 0)),
            pl.BlockSpec((TOPK, N_COLS), lambda c: (c, 0)),
        ],
        out_shape=[
            jax.ShapeDtypeStruct((N_ROWS, N_COLS), jnp.float32),
            jax.ShapeDtypeStruct((N_ROWS, N_COLS), jnp.int32),
        ],
    )(h)
    while sv.shape[0] > 128 * TOPK:
        npairs = sv.shape[0] // (2 * TOPK)
        sv, si = pl.pallas_call(
            _pairmerge_body,
            grid=(npairs,),
            in_specs=[
                pl.BlockSpec((2 * TOPK, N_COLS), lambda c: (c, 0)),
                pl.BlockSpec((2 * TOPK, N_COLS), lambda c: (c, 0)),
            ],
            out_specs=[
                pl.BlockSpec((TOPK, N_COLS), lambda c: (c, 0)),
                pl.BlockSpec((TOPK, N_COLS), lambda c: (c, 0)),
            ],
            out_shape=[
                jax.ShapeDtypeStruct((npairs * TOPK, N_COLS), jnp.float32),
                jax.ShapeDtypeStruct((npairs * TOPK, N_COLS), jnp.int32),
            ],
        )(sv, si)
    return pl.pallas_call(
        _merge_tail_body,
        out_shape=jax.ShapeDtypeStruct((TOPK, N_COLS), jnp.int32),
    )(sv, si)


def _sc_gather(h, idx):
    info = plsc.get_sparse_core_info()
    nc, ns = info.num_cores, info.num_subcores
    nw = nc * ns
    rows_per_w = TOPK // nw

    @functools.partial(
        pl.kernel,
        mesh=plsc.VectorSubcoreMesh(core_axis_name="c", subcore_axis_name="s"),
        out_type=jax.ShapeDtypeStruct((TOPK * N_COLS, N_COLS), jnp.float32),
        scratch_types=[
            pltpu.VMEM((rows_per_w, N_COLS), jnp.int32),
            pltpu.VMEM((N_COLS, N_COLS), jnp.float32),
            pltpu.VMEM((N_COLS, N_COLS), jnp.float32),
            pltpu.SemaphoreType.DMA,
            pltpu.SemaphoreType.DMA,
        ],
    )
    def gk(h_hbm, idx_hbm, out_hbm, idx_v, rows_v0, rows_v1, sem0, sem1):
        wid = lax.axis_index("s") * nc + lax.axis_index("c")
        pltpu.sync_copy(idx_hbm.at[pl.ds(wid * rows_per_w, rows_per_w)], idx_v)
        bufs = (rows_v0, rows_v1)
        sems = (sem0, sem1)
        cps = [pltpu.async_copy(h_hbm.at[idx_v.at[0]], bufs[0], sems[0]), None]
        for j in range(rows_per_w):
            if j + 1 < rows_per_w:
                cps[(j + 1) % 2] = pltpu.async_copy(
                    h_hbm.at[idx_v.at[j + 1]], bufs[(j + 1) % 2], sems[(j + 1) % 2])
            cps[j % 2].wait()
            pltpu.sync_copy(
                bufs[j % 2],
                out_hbm.at[pl.ds((wid * rows_per_w + j) * N_COLS, N_COLS)],
            )

    return gk(h, idx)


def kernel(h):
    idx = _topk_indices(h)
    flat = _sc_gather(h, idx)
    return flat.reshape(TOPK, N_COLS, N_COLS)


# speedup vs baseline: 1.1290x; 1.0002x over previous
import functools

import jax
import jax.numpy as jnp
from jax import lax
from jax.experimental import pallas as pl
from jax.experimental.pallas import tpu as pltpu
from jax.experimental.pallas import tpu_sc as plsc

N_ROWS = 32768
N_COLS = 128
TOPK = 256


def _roll(x, shift):
    return pltpu.roll(x, shift % x.shape[0], 0)


def _cmpx(v, i, dlog2, desc):
    d = 1 << dlog2
    vd = _roll(v, -d)
    vu = _roll(v, d)
    ii = _roll(i, -d)
    iu = _roll(i, d)
    rows = lax.broadcasted_iota(jnp.int32, (v.shape[0], 1), 0)
    low = ((rows >> dlog2) & 1) == 0
    pv = jnp.where(low, vd, vu)
    pi = jnp.where(low, ii, iu)
    self_gt = (v > pv) | ((v == pv) & (i < pi))
    keep_self = (low == desc) == self_gt
    return jnp.where(keep_self, v, pv), jnp.where(keep_self, i, pi)


def _cmpx_split(v, i, dlog2, s, row_offset=0, fd=None):
    d = 1 << dlog2
    r = v.shape[0]
    nb = r // (2 * d)
    v4 = v.reshape(nb, 2, d, N_COLS)
    i4 = i.reshape(nb, 2, d, N_COLS)
    av, bv = v4[:, 0], v4[:, 1]
    ai, bi = i4[:, 0], i4[:, 1]
    gt = (av > bv) | ((av == bv) & (ai < bi))
    mxv = jnp.where(gt, av, bv)
    mnv = jnp.where(gt, bv, av)
    mxi = jnp.where(gt, ai, bi)
    mni = jnp.where(gt, bi, ai)
    if fd is None:
        blk = lax.broadcasted_iota(jnp.int32, (nb, 1, 1), 0)
        desc = (((blk * 2 * d + row_offset) >> s) & 1) == 0
    else:
        desc = jnp.full((nb, 1, 1), fd)
    o0v = jnp.where(desc, mxv, mnv)
    o1v = jnp.where(desc, mnv, mxv)
    o0i = jnp.where(desc, mxi, mni)
    o1i = jnp.where(desc, mni, mxi)
    v2 = jnp.stack([o0v, o1v], axis=1).reshape(r, N_COLS)
    i2 = jnp.stack([o0i, o1i], axis=1).reshape(r, N_COLS)
    return v2, i2


def _halve(v, i, final_desc=None):
    av, bv = v[:TOPK], v[TOPK:]
    ai, bi = i[:TOPK], i[TOPK:]
    self_gt = (av > bv) | ((av == bv) & (ai < bi))
    v2 = jnp.where(self_gt, av, bv)
    i2 = jnp.where(self_gt, ai, bi)
    desc = jnp.full((TOPK, 1), final_desc)
    for j in range(7, 2, -1):
        v2, i2 = _cmpx_split(v2, i2, j, 8, fd=final_desc)
    for j in range(2, -1, -1):
        v2, i2 = _cmpx(v2, i2, j, desc)
    return v2, i2


def _sort_body(h_ref, v_out, i_out):
    c = pl.program_id(0)
    v = h_ref[...]
    i = lax.broadcasted_iota(jnp.int32, (TOPK, N_COLS), 0) + c * TOPK
    rows = lax.broadcasted_iota(jnp.int32, (TOPK, 1), 0) + c * TOPK
    for s in range(1, 9):
        desc = ((rows >> s) & 1) == 0
        for j in range(s - 1, -1, -1):
            if j >= 3:
                v, i = _cmpx_split(v, i, j, s, row_offset=c * TOPK)
            else:
                v, i = _cmpx(v, i, j, desc)
    v_out[...] = v
    i_out[...] = i


def _merge_tail_body(v_ref, i_ref, i_out):
    v = v_ref[...]
    i = i_ref[...]
    while v.shape[0] > TOPK:
        r = v.shape[0]
        nb = r // (2 * TOPK)
        parts = []
        for k in range(nb):
            blk_v = v[k * 2 * TOPK:(k + 1) * 2 * TOPK]
            blk_i = i[k * 2 * TOPK:(k + 1) * 2 * TOPK]
            parts.append(_halve(blk_v, blk_i, final_desc=(k % 2) == 0))
        if nb > 1:
            v = jnp.concatenate([p[0] for p in parts], axis=0)
            i = jnp.concatenate([p[1] for p in parts], axis=0)
        else:
            v, i = parts[0]
    i_out[...] = i


def _topk_indices(h):
    sv, si = pl.pallas_call(
        _sort_body,
        grid=(N_ROWS // TOPK,),
        in_specs=[pl.BlockSpec((TOPK, N_COLS), lambda c: (c, 0))],
        out_specs=[
            pl.BlockSpec((TOPK, N_COLS), lambda c: (c, 0)),
            pl.BlockSpec((TOPK, N_COLS), lambda c: (c, 0)),
        ],
        out_shape=[
            jax.ShapeDtypeStruct((N_ROWS, N_COLS), jnp.float32),
            jax.ShapeDtypeStruct((N_ROWS, N_COLS), jnp.int32),
        ],
    )(h)
    return pl.pallas_call(
        _merge_tail_body,
        out_shape=jax.ShapeDtypeStruct((TOPK, N_COLS), jnp.int32),
    )(sv, si)


def _sc_gather(h, idx):
    info = plsc.get_sparse_core_info()
    nc, ns = info.num_cores, info.num_subcores
    nw = nc * ns
    rows_per_w = TOPK // nw

    @functools.partial(
        pl.kernel,
        mesh=plsc.VectorSubcoreMesh(core_axis_name="c", subcore_axis_name="s"),
        out_type=jax.ShapeDtypeStruct((TOPK * N_COLS, N_COLS), jnp.float32),
        scratch_types=[
            pltpu.VMEM((rows_per_w, N_COLS), jnp.int32),
            pltpu.VMEM((N_COLS, N_COLS), jnp.float32),
            pltpu.VMEM((N_COLS, N_COLS), jnp.float32),
            pltpu.SemaphoreType.DMA,
            pltpu.SemaphoreType.DMA,
        ],
    )
    def gk(h_hbm, idx_hbm, out_hbm, idx_v, rows_v0, rows_v1, sem0, sem1):
        wid = lax.axis_index("s") * nc + lax.axis_index("c")
        pltpu.sync_copy(idx_hbm.at[pl.ds(wid * rows_per_w, rows_per_w)], idx_v)
        bufs = (rows_v0, rows_v1)
        sems = (sem0, sem1)
        cps = [pltpu.async_copy(h_hbm.at[idx_v.at[0]], bufs[0], sems[0]), None]
        for j in range(rows_per_w):
            if j + 1 < rows_per_w:
                cps[(j + 1) % 2] = pltpu.async_copy(
                    h_hbm.at[idx_v.at[j + 1]], bufs[(j + 1) % 2], sems[(j + 1) % 2])
            cps[j % 2].wait()
            pltpu.sync_copy(
                bufs[j % 2],
                out_hbm.at[pl.ds((wid * rows_per_w + j) * N_COLS, N_COLS)],
            )

    return gk(h, idx)


def kernel(h):
    idx = _topk_indices(h)
    flat = _sc_gather(h, idx)
    return flat.reshape(TOPK, N_COLS, N_COLS)
